# Initial kernel scaffold; baseline (speedup 1.0000x reference)
import functools

import jax
import jax.numpy as jnp
from jax import lax
from jax.experimental import pallas as pl
from jax.experimental.pallas import tpu as pltpu
from jax.experimental.pallas import tpu_sc as plsc

N = 10000
E = 320000
D = 128
NPAD = 10240

NCORES = 2
NSUB = 16
NW = NCORES * NSUB
CHUNK = 128
CPW = 80
EPAD = NW * CPW * CHUNK
ROWS_PER_SUB = NPAD // NSUB


def _tc1_body(nf_ref, lat_ref, wne_ref, wmsg_ref, ne_ref, a_ref, b_ref):
    nf = nf_ref[...] * (1.0 / 20000.0)
    lat = lat_ref[...]
    w0 = wne_ref[0:1, :]
    w1 = wne_ref[1:, :]
    ne = jnp.maximum(
        nf * w0 + jnp.dot(lat, w1, preferred_element_type=jnp.float32,
                          precision=lax.Precision.HIGHEST),
        0.0,
    )
    ne_ref[...] = ne
    a_ref[...] = jnp.dot(ne, wmsg_ref[:D, :], preferred_element_type=jnp.float32,
                         precision=lax.Precision.HIGHEST)
    b_ref[...] = jnp.dot(ne, wmsg_ref[D:, :], preferred_element_type=jnp.float32,
                         precision=lax.Precision.HIGHEST)


_tc1 = pl.pallas_call(
    _tc1_body,
    out_shape=(
        jax.ShapeDtypeStruct((NPAD, D), jnp.float32),
        jax.ShapeDtypeStruct((NPAD, D), jnp.float32),
        jax.ShapeDtypeStruct((NPAD, D), jnp.float32),
    ),
)


def _sc_body(a_hbm, b_hbm, src_hbm, dst_hbm, out_hbm,
             src_v, dst_v, ra_v, rb_v, agg_sh, sem_a, sem_b):
    cid = lax.axis_index("c")
    sid = lax.axis_index("s")
    gw = cid * NSUB + sid

    pltpu.sync_copy(src_hbm.at[gw], src_v)
    pltpu.sync_copy(dst_hbm.at[gw], dst_v)

    zeros16 = jnp.zeros((16,), jnp.float32)

    def _zbody(i, carry):
        r = i // (D // 16)
        c = (i % (D // 16)) * 16
        ra_v[r, pl.ds(c, 16)] = zeros16
        return carry

    lax.fori_loop(0, CHUNK * D // 16, _zbody, 0, unroll=8)
    for j in range(ROWS_PER_SUB // CHUNK):
        pltpu.sync_copy(ra_v, agg_sh.at[pl.ds(sid * ROWS_PER_SUB + j * CHUNK, CHUNK)])
    plsc.subcore_barrier()

    def _chunk_body(ch, carry):
        ca = pltpu.async_copy(a_hbm.at[src_v.at[ch]], ra_v, sem_a)
        cb = pltpu.async_copy(b_hbm.at[dst_v.at[ch]], rb_v, sem_b)
        ca.wait()
        cb.wait()

        def _cbody(i, c2):
            r = i // (D // 16)
            c = (i % (D // 16)) * 16
            ra_v[r, pl.ds(c, 16)] = jnp.maximum(
                ra_v[r, pl.ds(c, 16)] + rb_v[r, pl.ds(c, 16)], 0.0)
            return c2

        lax.fori_loop(0, CHUNK * D // 16, _cbody, 0, unroll=8)
        pltpu.sync_copy(ra_v, agg_sh.at[dst_v.at[ch]], add=True)
        return carry

    lax.fori_loop(0, CPW, _chunk_body, 0)
    plsc.subcore_barrier()

    pltpu.sync_copy(agg_sh.at[pl.ds(sid * ROWS_PER_SUB, ROWS_PER_SUB)],
                    out_hbm.at[cid, pl.ds(sid * ROWS_PER_SUB, ROWS_PER_SUB)])


_sc_agg = functools.partial(
    pl.kernel,
    out_type=jax.ShapeDtypeStruct((NCORES, NPAD, D), jnp.float32),
    mesh=plsc.VectorSubcoreMesh(core_axis_name="c", subcore_axis_name="s"),
    scratch_types=[
        pltpu.VMEM((CPW, CHUNK), jnp.int32),
        pltpu.VMEM((CPW, CHUNK), jnp.int32),
        pltpu.VMEM((CHUNK, D), jnp.float32),
        pltpu.VMEM((CHUNK, D), jnp.float32),
        pltpu.VMEM_SHARED((NPAD, D), jnp.float32),
        pltpu.SemaphoreType.DMA,
        pltpu.SemaphoreType.DMA,
    ],
)(_sc_body)


def _tc2_body(ne_ref, agg_ref, wupd_ref, wdec_ref, out_ref, lo_ref):
    ne = ne_ref[...]
    agg = agg_ref[0] + agg_ref[1]
    lo = jnp.maximum(
        jnp.dot(ne, wupd_ref[:D, :], preferred_element_type=jnp.float32,
                precision=lax.Precision.HIGHEST)
        + jnp.dot(agg, wupd_ref[D:, :], preferred_element_type=jnp.float32,
                  precision=lax.Precision.HIGHEST),
        0.0,
    )
    lo_ref[...] = lo
    out_ref[...] = (
        jnp.dot(ne, wdec_ref[:D, :], preferred_element_type=jnp.float32,
                precision=lax.Precision.HIGHEST)
        + jnp.dot(lo, wdec_ref[D:, :], preferred_element_type=jnp.float32,
                  precision=lax.Precision.HIGHEST)
    )


_tc2 = pl.pallas_call(
    _tc2_body,
    out_shape=(
        jax.ShapeDtypeStruct((NPAD, 1), jnp.float32),
        jax.ShapeDtypeStruct((NPAD, D), jnp.float32),
    ),
)


def kernel(read_length, overlap_similarity, latent_features, edge_index,
           W_node_enc, W_edge_enc, W_msg, W_upd, W_dec):
    del overlap_similarity, W_edge_enc

    nf = jnp.zeros((NPAD, 1), jnp.float32).at[:N, 0].set(read_length)
    lat = jnp.zeros((NPAD, D), jnp.float32).at[:N].set(latent_features)

    ne, a, b = _tc1(nf, lat, W_node_enc, W_msg)

    src = edge_index[0].astype(jnp.int32)
    dst = edge_index[1].astype(jnp.int32)
    pad = jnp.full((EPAD - E,), N, jnp.int32)
    src3 = jnp.concatenate([src, pad]).reshape(NW, CPW, CHUNK)
    dst3 = jnp.concatenate([dst, pad]).reshape(NW, CPW, CHUNK)

    agg2 = _sc_agg(a, b, src3, dst3)

    out, lo = _tc2(ne, agg2, W_upd, W_dec)
    return (out[:N], lo[:N])


# speedup vs baseline: 2.3302x; 2.3302x over previous
import functools

import jax
import jax.numpy as jnp
from jax import lax
from jax.experimental import pallas as pl
from jax.experimental.pallas import tpu as pltpu
from jax.experimental.pallas import tpu_sc as plsc

N = 10000
E = 320000
D = 128
NPAD = 10240

NCORES = 2
NSUB = 16
NW = NCORES * NSUB
CHUNK = 128
CPW = 80
GRP = 8
EPAD = NW * CPW * CHUNK
ROWS_PER_SUB = NPAD // NSUB


def _tc1_body(nf_ref, lat_ref, wne_ref, wmsg_ref, ne_ref, a_ref, b_ref):
    nf = nf_ref[...] * (1.0 / 20000.0)
    lat = lat_ref[...]
    w0 = wne_ref[0:1, :]
    w1 = wne_ref[1:, :]
    ne = jnp.maximum(
        nf * w0 + jnp.dot(lat, w1, preferred_element_type=jnp.float32,
                          precision=lax.Precision.HIGHEST),
        0.0,
    )
    ne_ref[...] = ne
    a_ref[...] = jnp.dot(ne, wmsg_ref[:D, :], preferred_element_type=jnp.float32,
                         precision=lax.Precision.HIGHEST)
    b_ref[...] = jnp.dot(ne, wmsg_ref[D:, :], preferred_element_type=jnp.float32,
                         precision=lax.Precision.HIGHEST)


_tc1 = pl.pallas_call(
    _tc1_body,
    out_shape=(
        jax.ShapeDtypeStruct((NPAD, D), jnp.float32),
        jax.ShapeDtypeStruct((NPAD, D), jnp.float32),
        jax.ShapeDtypeStruct((NPAD, D), jnp.float32),
    ),
)


def _sc_body(a_hbm, b_hbm, src_hbm, dst_hbm, out_hbm,
             src_v, dst_v, ra_v, rb_v, agg_sh, sem_a, sem_b):
    cid = lax.axis_index("c")
    sid = lax.axis_index("s")
    gw = cid * NSUB + sid

    zeros16 = jnp.zeros((16,), jnp.float32)

    def _zbody(i, carry):
        r = i // (D // 16)
        c = (i % (D // 16)) * 16
        ra_v[r, pl.ds(c, 16)] = zeros16
        return carry

    lax.fori_loop(0, CHUNK * D // 16, _zbody, 0, unroll=8)
    for j in range(ROWS_PER_SUB // CHUNK):
        pltpu.sync_copy(ra_v, agg_sh.at[pl.ds(sid * ROWS_PER_SUB + j * CHUNK, CHUNK)])
    plsc.subcore_barrier()

    def _group_body(g, carry):
        pltpu.sync_copy(src_hbm.at[gw, pl.ds(g * GRP, GRP)], src_v)
        pltpu.sync_copy(dst_hbm.at[gw, pl.ds(g * GRP, GRP)], dst_v)

        def _chunk_body(ch, c1):
            ca = pltpu.async_copy(a_hbm.at[src_v.at[ch]], ra_v, sem_a)
            cb = pltpu.async_copy(b_hbm.at[dst_v.at[ch]], rb_v, sem_b)
            ca.wait()
            cb.wait()

            def _cbody(i, c2):
                r = i // (D // 16)
                c = (i % (D // 16)) * 16
                ra_v[r, pl.ds(c, 16)] = jnp.maximum(
                    ra_v[r, pl.ds(c, 16)] + rb_v[r, pl.ds(c, 16)], 0.0)
                return c2

            lax.fori_loop(0, CHUNK * D // 16, _cbody, 0, unroll=8)
            pltpu.sync_copy(ra_v, agg_sh.at[dst_v.at[ch]], add=True)
            return c1

        lax.fori_loop(0, GRP, _chunk_body, 0)
        return carry

    lax.fori_loop(0, CPW // GRP, _group_body, 0)
    plsc.subcore_barrier()

    pltpu.sync_copy(agg_sh.at[pl.ds(sid * ROWS_PER_SUB, ROWS_PER_SUB)],
                    out_hbm.at[cid, pl.ds(sid * ROWS_PER_SUB, ROWS_PER_SUB)])


_sc_agg = functools.partial(
    pl.kernel,
    out_type=jax.ShapeDtypeStruct((NCORES, NPAD, D), jnp.float32),
    mesh=plsc.VectorSubcoreMesh(core_axis_name="c", subcore_axis_name="s"),
    scratch_types=[
        pltpu.VMEM((GRP, CHUNK), jnp.int32),
        pltpu.VMEM((GRP, CHUNK), jnp.int32),
        pltpu.VMEM((CHUNK, D), jnp.float32),
        pltpu.VMEM((CHUNK, D), jnp.float32),
        pltpu.VMEM_SHARED((NPAD, D), jnp.float32),
        pltpu.SemaphoreType.DMA,
        pltpu.SemaphoreType.DMA,
    ],
)(_sc_body)


def _tc2_body(ne_ref, agg_ref, wupd_ref, wdec_ref, out_ref, lo_ref):
    ne = ne_ref[...]
    agg = agg_ref[0] + agg_ref[1]
    lo = jnp.maximum(
        jnp.dot(ne, wupd_ref[:D, :], preferred_element_type=jnp.float32,
                precision=lax.Precision.HIGHEST)
        + jnp.dot(agg, wupd_ref[D:, :], preferred_element_type=jnp.float32,
                  precision=lax.Precision.HIGHEST),
        0.0,
    )
    lo_ref[...] = lo
    out_ref[...] = (
        jnp.dot(ne, wdec_ref[:D, :], preferred_element_type=jnp.float32,
                precision=lax.Precision.HIGHEST)
        + jnp.dot(lo, wdec_ref[D:, :], preferred_element_type=jnp.float32,
                  precision=lax.Precision.HIGHEST)
    )


_RB = 1024

_tc2 = pl.pallas_call(
    _tc2_body,
    grid=(NPAD // _RB,),
    in_specs=[
        pl.BlockSpec((_RB, D), lambda i: (i, 0)),
        pl.BlockSpec((NCORES, _RB, D), lambda i: (0, i, 0)),
        pl.BlockSpec((2 * D, D), lambda i: (0, 0)),
        pl.BlockSpec((2 * D, 1), lambda i: (0, 0)),
    ],
    out_specs=(
        pl.BlockSpec((_RB, 1), lambda i: (i, 0)),
        pl.BlockSpec((_RB, D), lambda i: (i, 0)),
    ),
    out_shape=(
        jax.ShapeDtypeStruct((NPAD, 1), jnp.float32),
        jax.ShapeDtypeStruct((NPAD, D), jnp.float32),
    ),
)


def kernel(read_length, overlap_similarity, latent_features, edge_index,
           W_node_enc, W_edge_enc, W_msg, W_upd, W_dec):
    del overlap_similarity, W_edge_enc

    nf = jnp.zeros((NPAD, 1), jnp.float32).at[:N, 0].set(read_length)
    lat = jnp.zeros((NPAD, D), jnp.float32).at[:N].set(latent_features)

    ne, a, b = _tc1(nf, lat, W_node_enc, W_msg)

    src = edge_index[0].astype(jnp.int32)
    dst = edge_index[1].astype(jnp.int32)
    pad = jnp.full((EPAD - E,), N, jnp.int32)
    src3 = jnp.concatenate([src, pad]).reshape(NW, CPW, CHUNK)
    dst3 = jnp.concatenate([dst, pad]).reshape(NW, CPW, CHUNK)

    agg2 = _sc_agg(a, b, src3, dst3)

    out, lo = _tc2(ne, agg2, W_upd, W_dec)
    return (out[:N], lo[:N])


# speedup vs baseline: 2.8642x; 1.2292x over previous
import functools

import jax
import jax.numpy as jnp
from jax import lax
from jax.experimental import pallas as pl
from jax.experimental.pallas import tpu as pltpu
from jax.experimental.pallas import tpu_sc as plsc

N = 10000
E = 320000
D = 128
NPAD = 10240

NCORES = 2
NSUB = 16
NW = NCORES * NSUB
CHUNK = 64
CPW = 160
GI = 20
NGI = CPW // GI
EPAD = NW * CPW * CHUNK
ROWS_PER_SUB = NPAD // NSUB


def _tc1_body(nf_ref, lat_ref, wne_ref, wmsg_ref, ne_ref, a_ref, b_ref):
    nf = nf_ref[...] * (1.0 / 20000.0)
    lat = lat_ref[...]
    w0 = wne_ref[0:1, :]
    w1 = wne_ref[1:, :]
    ne = jnp.maximum(
        nf * w0 + jnp.dot(lat, w1, preferred_element_type=jnp.float32,
                          precision=lax.Precision.HIGHEST),
        0.0,
    )
    ne_ref[...] = ne
    a_ref[...] = jnp.dot(ne, wmsg_ref[:D, :], preferred_element_type=jnp.float32,
                         precision=lax.Precision.HIGHEST)
    b_ref[...] = jnp.dot(ne, wmsg_ref[D:, :], preferred_element_type=jnp.float32,
                         precision=lax.Precision.HIGHEST)


_tc1 = pl.pallas_call(
    _tc1_body,
    out_shape=(
        jax.ShapeDtypeStruct((NPAD, D), jnp.float32),
        jax.ShapeDtypeStruct((NPAD, D), jnp.float32),
        jax.ShapeDtypeStruct((NPAD, D), jnp.float32),
    ),
)


def _sc_body(a_hbm, b_hbm, src_hbm, dst_hbm, out_hbm,
             src_v, dst_v, ra_v, rb_v, agg_sh,
             sem_g0, sem_g1, sem_s0, sem_s1):
    cid = lax.axis_index("c")
    sid = lax.axis_index("s")
    gw = cid * NSUB + sid
    sem_g = (sem_g0, sem_g1)
    sem_s = (sem_s0, sem_s1)

    zeros16 = jnp.zeros((16,), jnp.float32)

    def _zbody(i, carry):
        r = i // (D // 16)
        c = (i % (D // 16)) * 16
        ra_v[0, r, pl.ds(c, 16)] = zeros16
        return carry

    lax.fori_loop(0, CHUNK * D // 16, _zbody, 0, unroll=8)
    for j in range(ROWS_PER_SUB // CHUNK):
        pltpu.sync_copy(ra_v.at[0],
                        agg_sh.at[pl.ds(sid * ROWS_PER_SUB + j * CHUNK, CHUNK)])
    plsc.subcore_barrier()

    def _gathers(ch, p):
        slab = (ch // GI) % 2
        row = ch % GI
        ca = pltpu.make_async_copy(a_hbm.at[src_v.at[slab, row]], ra_v.at[p],
                                   sem_g[p])
        cb = pltpu.make_async_copy(b_hbm.at[dst_v.at[slab, row]], rb_v.at[p],
                                   sem_g[p])
        return ca, cb

    def _scatter(ch, p):
        slab = (ch // GI) % 2
        row = ch % GI
        return pltpu.make_async_copy(ra_v.at[p], agg_sh.at[dst_v.at[slab, row]],
                                     sem_s[p])

    pltpu.sync_copy(src_hbm.at[gw, 0], src_v.at[0])
    pltpu.sync_copy(dst_hbm.at[gw, 0], dst_v.at[0])
    ca, cb = _gathers(0, 0)
    ca.start()
    cb.start()

    def _pair_body(k, carry):
        ch0 = 2 * k
        g = ch0 // GI

        @pl.when(jnp.logical_and(ch0 % GI == 0, g + 1 < NGI))
        def _():
            pltpu.sync_copy(src_hbm.at[gw, g + 1], src_v.at[(g + 1) % 2])
            pltpu.sync_copy(dst_hbm.at[gw, g + 1], dst_v.at[(g + 1) % 2])

        for p in range(2):
            ch = ch0 + p

            @pl.when(ch > 0)
            def _(p=p, ch=ch):
                _scatter(ch - 1, 1 - p).wait()

            @pl.when(ch + 1 < CPW)
            def _(p=p, ch=ch):
                na, nb = _gathers(ch + 1, 1 - p)
                na.start()
                nb.start()

            wa, wb = _gathers(ch, p)
            wa.wait()
            wb.wait()

            def _cbody(i, c2, p=p):
                r = i // (D // 16)
                c = (i % (D // 16)) * 16
                ra_v[p, r, pl.ds(c, 16)] = jnp.maximum(
                    ra_v[p, r, pl.ds(c, 16)] + rb_v[p, r, pl.ds(c, 16)], 0.0)
                return c2

            lax.fori_loop(0, CHUNK * D // 16, _cbody, 0, unroll=8)
            _scatter(ch, p).start(add=True)
        return carry

    lax.fori_loop(0, CPW // 2, _pair_body, 0)
    _scatter(CPW - 1, 1).wait()
    plsc.subcore_barrier()

    pltpu.sync_copy(agg_sh.at[pl.ds(sid * ROWS_PER_SUB, ROWS_PER_SUB)],
                    out_hbm.at[cid, pl.ds(sid * ROWS_PER_SUB, ROWS_PER_SUB)])


_sc_agg = functools.partial(
    pl.kernel,
    out_type=jax.ShapeDtypeStruct((NCORES, NPAD, D), jnp.float32),
    mesh=plsc.VectorSubcoreMesh(core_axis_name="c", subcore_axis_name="s"),
    scratch_types=[
        pltpu.VMEM((2, GI, CHUNK), jnp.int32),
        pltpu.VMEM((2, GI, CHUNK), jnp.int32),
        pltpu.VMEM((2, CHUNK, D), jnp.float32),
        pltpu.VMEM((2, CHUNK, D), jnp.float32),
        pltpu.VMEM_SHARED((NPAD, D), jnp.float32),
        pltpu.SemaphoreType.DMA,
        pltpu.SemaphoreType.DMA,
        pltpu.SemaphoreType.DMA,
        pltpu.SemaphoreType.DMA,
    ],
)(_sc_body)


def _tc2_body(ne_ref, agg_ref, wupd_ref, wdec_ref, out_ref, lo_ref):
    ne = ne_ref[...]
    agg = agg_ref[0] + agg_ref[1]
    lo = jnp.maximum(
        jnp.dot(ne, wupd_ref[:D, :], preferred_element_type=jnp.float32,
                precision=lax.Precision.HIGHEST)
        + jnp.dot(agg, wupd_ref[D:, :], preferred_element_type=jnp.float32,
                  precision=lax.Precision.HIGHEST),
        0.0,
    )
    lo_ref[...] = lo
    out_ref[...] = (
        jnp.dot(ne, wdec_ref[:D, :], preferred_element_type=jnp.float32,
                precision=lax.Precision.HIGHEST)
        + jnp.dot(lo, wdec_ref[D:, :], preferred_element_type=jnp.float32,
                  precision=lax.Precision.HIGHEST)
    )


_RB = 1024

_tc2 = pl.pallas_call(
    _tc2_body,
    grid=(NPAD // _RB,),
    in_specs=[
        pl.BlockSpec((_RB, D), lambda i: (i, 0)),
        pl.BlockSpec((NCORES, _RB, D), lambda i: (0, i, 0)),
        pl.BlockSpec((2 * D, D), lambda i: (0, 0)),
        pl.BlockSpec((2 * D, 1), lambda i: (0, 0)),
    ],
    out_specs=(
        pl.BlockSpec((_RB, 1), lambda i: (i, 0)),
        pl.BlockSpec((_RB, D), lambda i: (i, 0)),
    ),
    out_shape=(
        jax.ShapeDtypeStruct((NPAD, 1), jnp.float32),
        jax.ShapeDtypeStruct((NPAD, D), jnp.float32),
    ),
)


def kernel(read_length, overlap_similarity, latent_features, edge_index,
           W_node_enc, W_edge_enc, W_msg, W_upd, W_dec):
    del overlap_similarity, W_edge_enc

    nf = jnp.zeros((NPAD, 1), jnp.float32).at[:N, 0].set(read_length)
    lat = jnp.zeros((NPAD, D), jnp.float32).at[:N].set(latent_features)

    ne, a, b = _tc1(nf, lat, W_node_enc, W_msg)

    src = edge_index[0].astype(jnp.int32)
    dst = edge_index[1].astype(jnp.int32)
    pad = jnp.full((EPAD - E,), N, jnp.int32)
    src3 = jnp.concatenate([src, pad]).reshape(NW, NGI, GI, CHUNK)
    dst3 = jnp.concatenate([dst, pad]).reshape(NW, NGI, GI, CHUNK)

    agg2 = _sc_agg(a, b, src3, dst3)

    out, lo = _tc2(ne, agg2, W_upd, W_dec)
    return (out[:N], lo[:N])
